# baseline (speedup 1.0000x reference)
import functools

import jax
import jax.numpy as jnp
from jax.experimental import pallas as pl
from jax.experimental.pallas import tpu as pltpu

STATE_DIM = 256
POCKET_DIM = 64
MAX_STATES = 64
MIN_STATES = 16
B = 1024
N_POCKET = 512
NP = N_POCKET * POCKET_DIM

B_TILE = 128
NB = B // B_TILE


P_SPLIT = 2
P_HALF = POCKET_DIM // P_SPLIT
NP_HALF = NP // P_SPLIT


def _fused_kernel(mol, base_t, sW1, sb1, sW2t, sb2, dW1, db1_t, dW2t,
                  db2_t, probs_t_out, wp_t_out, s2t, p_sc):
    i = pl.program_id(0)
    j = pl.program_id(1)

    @pl.when(jnp.logical_and(i == 0, j == 0))
    def _():
        bt = base_t[...]
        for k in range(MIN_STATES):
            h1_t = jax.nn.silu(
                jax.lax.dot_general(
                    dW1[k], bt, (((0,), (0,)), ((), ())),
                    preferred_element_type=jnp.float32)
                + db1_t[:, k:k + 1])
            d_t = (jax.lax.dot_general(
                dW2t[k], h1_t, (((1,), (0,)), ((), ())),
                preferred_element_type=jnp.float32)
                + db2_t[:, k:k + 1])
            st = bt + 0.1 * d_t
            for hh in range(P_SPLIT):
                s2t[hh, k:k + 1, :] = st[hh * P_HALF:(hh + 1) * P_HALF,
                                         :].reshape(1, NP_HALF)

    @pl.when(j == 0)
    def _():
        h = jax.nn.silu(
            jnp.dot(mol[...], sW1[...],
                    preferred_element_type=jnp.float32) + sb1[...])
        logits = (jax.lax.dot_general(
            h, sW2t[...], (((1,), (1,)), ((), ())),
            preferred_element_type=jnp.float32) + sb2[...])
        col = jax.lax.broadcasted_iota(jnp.int32, logits.shape, 1)
        masked = jnp.where(col < MIN_STATES, logits, -jnp.inf)
        m = jnp.max(masked, axis=1, keepdims=True)
        e = jnp.exp(masked - m)
        p = e / jnp.sum(e, axis=1, keepdims=True)
        p_sc[...] = p
        probs_t_out[...] = jnp.transpose(p)

    rhs = s2t[j]
    res = jnp.dot(p_sc[:, :MIN_STATES], rhs,
                  preferred_element_type=jnp.float32)
    wp_t_out[...] = res.reshape(wp_t_out.shape)


@functools.partial(jax.jit, static_argnames=())
def kernel(mol_embedding, base_pocket, sel_W1, sel_b1, sel_W2, sel_b2,
           def_W1, def_b1, def_W2, def_b2):
    probs_t, wp_t = pl.pallas_call(
        _fused_kernel,
        grid=(NB, P_SPLIT),
        in_specs=[
            pl.BlockSpec((B_TILE, STATE_DIM), lambda i, j: (i, 0)),
            pl.BlockSpec((POCKET_DIM, N_POCKET), lambda i, j: (0, 0)),
            pl.BlockSpec((STATE_DIM, STATE_DIM), lambda i, j: (0, 0)),
            pl.BlockSpec((1, STATE_DIM), lambda i, j: (0, 0)),
            pl.BlockSpec((MAX_STATES, STATE_DIM), lambda i, j: (0, 0)),
            pl.BlockSpec((1, MAX_STATES), lambda i, j: (0, 0)),
            pl.BlockSpec((MIN_STATES, POCKET_DIM, STATE_DIM),
                         lambda i, j: (0, 0, 0)),
            pl.BlockSpec((STATE_DIM, MIN_STATES), lambda i, j: (0, 0)),
            pl.BlockSpec((MIN_STATES, POCKET_DIM, STATE_DIM),
                         lambda i, j: (0, 0, 0)),
            pl.BlockSpec((POCKET_DIM, MIN_STATES), lambda i, j: (0, 0)),
        ],
        out_specs=[
            pl.BlockSpec((MAX_STATES, B_TILE), lambda i, j: (0, i)),
            pl.BlockSpec((B_TILE, P_HALF, N_POCKET),
                         lambda i, j: (i, j, 0)),
        ],
        out_shape=[
            jax.ShapeDtypeStruct((MAX_STATES, B), jnp.float32),
            jax.ShapeDtypeStruct((B, POCKET_DIM, N_POCKET), jnp.float32),
        ],
        scratch_shapes=[
            pltpu.VMEM((P_SPLIT, MIN_STATES, NP_HALF), jnp.float32),
            pltpu.VMEM((B_TILE, MAX_STATES), jnp.float32),
        ],
        compiler_params=pltpu.CompilerParams(
            dimension_semantics=("arbitrary", "arbitrary")),
    )(mol_embedding, base_pocket.T, sel_W1, sel_b1.reshape(1, -1),
      sel_W2.T, sel_b2.reshape(1, -1), def_W1, def_b1.T,
      def_W2.transpose(0, 2, 1), def_b2.T)

    weighted_pocket = wp_t.transpose(0, 2, 1)
    probs = probs_t.T
    active_indices = jnp.arange(MIN_STATES, dtype=jnp.int32)
    return weighted_pocket, probs, active_indices


# speedup vs baseline: 1.0428x; 1.0428x over previous
import functools

import jax
import jax.numpy as jnp
from jax.experimental import pallas as pl
from jax.experimental.pallas import tpu as pltpu

STATE_DIM = 256
POCKET_DIM = 64
MAX_STATES = 64
MIN_STATES = 16
B = 1024
N_POCKET = 512
NP = N_POCKET * POCKET_DIM

B_TILE = 128
NB = B // B_TILE


def _fused_kernel(mol, base_t, sW1, sb1, sW2t, sb2, dW1, db1_t, dW2t,
                  db2_t, probs_t_out, wp_t_out, s2t):
    i = pl.program_id(0)

    @pl.when(i == 0)
    def _():
        bt = base_t[...]
        for k in range(MIN_STATES):
            h1_t = jax.nn.silu(
                jax.lax.dot_general(
                    dW1[k], bt, (((0,), (0,)), ((), ())),
                    preferred_element_type=jnp.float32)
                + db1_t[:, k:k + 1])
            d_t = (jax.lax.dot_general(
                dW2t[k], h1_t, (((1,), (0,)), ((), ())),
                preferred_element_type=jnp.float32)
                + db2_t[:, k:k + 1])
            s2t[k, :, :] = bt + 0.1 * d_t

    h = jax.nn.silu(
        jnp.dot(mol[...], sW1[...], preferred_element_type=jnp.float32)
        + sb1[...])
    logits = (jax.lax.dot_general(
        h, sW2t[...], (((1,), (1,)), ((), ())),
        preferred_element_type=jnp.float32) + sb2[...])
    col = jax.lax.broadcasted_iota(jnp.int32, logits.shape, 1)
    masked = jnp.where(col < MIN_STATES, logits, -jnp.inf)
    m = jnp.max(masked, axis=1, keepdims=True)
    e = jnp.exp(masked - m)
    p = e / jnp.sum(e, axis=1, keepdims=True)
    probs_t_out[...] = jnp.transpose(p)

    rhs = s2t[...].reshape(MIN_STATES, NP)
    res = jnp.dot(p[:, :MIN_STATES], rhs,
                  preferred_element_type=jnp.float32)
    wp_t_out[...] = res.reshape(wp_t_out.shape)


@functools.partial(jax.jit, static_argnames=())
def kernel(mol_embedding, base_pocket, sel_W1, sel_b1, sel_W2, sel_b2,
           def_W1, def_b1, def_W2, def_b2):
    probs_t, wp_t = pl.pallas_call(
        _fused_kernel,
        grid=(NB,),
        in_specs=[
            pl.BlockSpec((B_TILE, STATE_DIM), lambda i: (i, 0)),
            pl.BlockSpec((POCKET_DIM, N_POCKET), lambda i: (0, 0)),
            pl.BlockSpec((STATE_DIM, STATE_DIM), lambda i: (0, 0)),
            pl.BlockSpec((1, STATE_DIM), lambda i: (0, 0)),
            pl.BlockSpec((MAX_STATES, STATE_DIM), lambda i: (0, 0)),
            pl.BlockSpec((1, MAX_STATES), lambda i: (0, 0)),
            pl.BlockSpec((MIN_STATES, POCKET_DIM, STATE_DIM),
                         lambda i: (0, 0, 0)),
            pl.BlockSpec((STATE_DIM, MIN_STATES), lambda i: (0, 0)),
            pl.BlockSpec((MIN_STATES, POCKET_DIM, STATE_DIM),
                         lambda i: (0, 0, 0)),
            pl.BlockSpec((POCKET_DIM, MIN_STATES), lambda i: (0, 0)),
        ],
        out_specs=[
            pl.BlockSpec((MAX_STATES, B_TILE), lambda i: (0, i)),
            pl.BlockSpec((B_TILE, POCKET_DIM, N_POCKET),
                         lambda i: (i, 0, 0)),
        ],
        out_shape=[
            jax.ShapeDtypeStruct((MAX_STATES, B), jnp.float32),
            jax.ShapeDtypeStruct((B, POCKET_DIM, N_POCKET), jnp.float32),
        ],
        scratch_shapes=[
            pltpu.VMEM((MIN_STATES, POCKET_DIM, N_POCKET), jnp.float32),
        ],
        compiler_params=pltpu.CompilerParams(
            dimension_semantics=("arbitrary",)),
    )(mol_embedding, base_pocket.T, sel_W1, sel_b1.reshape(1, -1),
      sel_W2.T, sel_b2.reshape(1, -1), def_W1, def_b1.T,
      def_W2.transpose(0, 2, 1), def_b2.T)

    weighted_pocket = wp_t.transpose(0, 2, 1)
    probs = probs_t.T
    active_indices = jnp.arange(MIN_STATES, dtype=jnp.int32)
    return weighted_pocket, probs, active_indices
